# baseline (speedup 1.0000x reference)
import functools

import jax
import jax.numpy as jnp
from jax.experimental import pallas as pl
from jax.experimental.pallas import tpu as pltpu
from jax.experimental.pallas import tpu_sc as plsc

B = 2048
D = 2048
E = 8
K = 2
H = 4096

TM = 512
T = B * K // TM + E
NROWS = T * TM
TH = 1024
NH = H // TH

_NEG_INF = float("-inf")


def _router_kernel(x_ref, wgt_ref, bg_ref, gates_ref, pos0_ref, pos1_ref,
                   texp_ref, xsidx_ref, valid_ref):
    x = x_ref[...]
    logits = jax.lax.dot(x, wgt_ref[...],
                         preferred_element_type=jnp.float32)
    logits = logits + bg_ref[...]

    idx = jax.lax.broadcasted_iota(jnp.int32, (B, E), 1)
    m1 = jnp.max(logits, axis=1, keepdims=True)
    i1 = jnp.min(jnp.where(logits == m1, idx, E), axis=1, keepdims=True)
    masked = jnp.where(idx == i1, _NEG_INF, logits)
    m2 = jnp.max(masked, axis=1, keepdims=True)
    i2 = jnp.min(jnp.where(masked == m2, idx, E), axis=1, keepdims=True)

    e2 = jnp.exp(m2 - m1)
    denom = 1.0 + e2
    g1 = 1.0 / denom
    g2 = e2 / denom
    gates_ref[...] = jnp.concatenate([g1, g2], axis=1)

    oh1 = (idx == i1).astype(jnp.float32)
    oh2 = (idx == i2).astype(jnp.float32)
    mh = oh1 + oh2

    r_iota = jax.lax.broadcasted_iota(jnp.int32, (B, B), 0)
    c_iota = jax.lax.broadcasted_iota(jnp.int32, (B, B), 1)
    ltri = (c_iota < r_iota).astype(jnp.float32)
    csum = jax.lax.dot(ltri, mh, preferred_element_type=jnp.float32)

    counts = jnp.sum(mh, axis=0, keepdims=True)
    padded = jnp.ceil(counts / TM) * TM

    er = jax.lax.broadcasted_iota(jnp.int32, (E, E), 0)
    ec = jax.lax.broadcasted_iota(jnp.int32, (E, E), 1)
    utri = (er < ec).astype(jnp.float32)
    starts = jax.lax.dot(padded, utri,
                         preferred_element_type=jnp.float32)
    ends = starts + padded
    total = jnp.sum(padded, axis=1, keepdims=True)

    rank1 = jnp.sum(csum * oh1, axis=1, keepdims=True)
    rank2 = jnp.sum(csum * oh2, axis=1, keepdims=True)
    start1 = jnp.sum(starts * oh1, axis=1, keepdims=True)
    start2 = jnp.sum(starts * oh2, axis=1, keepdims=True)
    pos0_ref[...] = jnp.round(start1 + rank1).astype(jnp.int32)
    pos1_ref[...] = jnp.round(start2 + rank2).astype(jnp.int32)

    t_col = (jax.lax.broadcasted_iota(jnp.int32, (T, 1), 0) * TM
             ).astype(jnp.float32)
    t_cmp = jnp.sum((jnp.broadcast_to(ends, (T, E)) <=
                     jnp.broadcast_to(t_col, (T, E))).astype(jnp.int32),
                    axis=1, keepdims=True)
    texp_last = jnp.sum((ends <= (total - TM)).astype(jnp.int32),
                        axis=1, keepdims=True)
    texp_ref[...] = jnp.minimum(t_cmp, texp_last)
    n_last = jnp.round(total / TM).astype(jnp.int32) - 1
    t_idx = jax.lax.broadcasted_iota(jnp.int32, (T, 1), 0)
    xsidx_ref[...] = jnp.minimum(t_idx, n_last)
    valid_ref[...] = (t_idx <= n_last).astype(jnp.int32)


def _run_router(x, wgt, bg2d, *, interpret=False):
    out_shapes = (
        jax.ShapeDtypeStruct((B, K), jnp.float32),
        jax.ShapeDtypeStruct((B, 1), jnp.int32),
        jax.ShapeDtypeStruct((B, 1), jnp.int32),
        jax.ShapeDtypeStruct((T, 1), jnp.int32),
        jax.ShapeDtypeStruct((T, 1), jnp.int32),
        jax.ShapeDtypeStruct((T, 1), jnp.int32),
    )
    return pl.pallas_call(
        _router_kernel,
        out_shape=out_shapes,
        interpret=interpret,
    )(x, wgt, bg2d)


_NC = 2
_NS = 16
_NW = _NC * _NS
_CH = 16
_NCH = B // (_NW * _CH)


def _dispatch_sc(x, pos0, pos1):
    mesh = plsc.VectorSubcoreMesh(core_axis_name="c", subcore_axis_name="s")
    width = x.shape[1]

    @functools.partial(
        pl.kernel, mesh=mesh,
        out_type=jax.ShapeDtypeStruct((NROWS, width), x.dtype),
        scratch_types=[
            pltpu.VMEM((_CH,), jnp.int32),
            pltpu.VMEM((_CH,), jnp.int32),
            pltpu.VMEM((_CH, width), x.dtype),
        ],
    )
    def scatter_kernel(x_hbm, p0_hbm, p1_hbm, o_hbm, i0_v, i1_v, rows_v):
        wid = jax.lax.axis_index("s") * _NC + jax.lax.axis_index("c")

        @pl.loop(0, _NCH)
        def _(c):
            j = wid * _NCH + c
            base = j * _CH
            pltpu.sync_copy(p0_hbm.at[j], i0_v)
            pltpu.sync_copy(p1_hbm.at[j], i1_v)
            pltpu.sync_copy(x_hbm.at[pl.ds(base, _CH)], rows_v)
            pltpu.sync_copy(rows_v, o_hbm.at[i0_v])
            pltpu.sync_copy(rows_v, o_hbm.at[i1_v])

    return scatter_kernel(x, pos0, pos1)


def _mlp1_kernel(texp_ref, xsidx_ref, valid_ref,
                 xs_ref, w1_ref, b1_ref, hs_ref):
    t = pl.program_id(1)

    @pl.when(valid_ref[t] == 1)
    def _():
        xb = xs_ref[...]
        w1 = w1_ref[0].astype(jnp.bfloat16)
        hb = jax.lax.dot(xb, w1, preferred_element_type=jnp.float32)
        hb = jnp.maximum(hb + b1_ref[0], 0.0)
        hs_ref[...] = hb.astype(jnp.bfloat16)


def _run_mlp1(xs, w1, b1, texp, xsidx, valid, *, interpret=False):
    grid_spec = pltpu.PrefetchScalarGridSpec(
        num_scalar_prefetch=3,
        grid=(NH, T),
        in_specs=[
            pl.BlockSpec((TM, D), lambda h, t, te, xi, va: (xi[t], 0)),
            pl.BlockSpec((1, D, TH), lambda h, t, te, xi, va: (te[t], 0, h)),
            pl.BlockSpec((1, 1, TH), lambda h, t, te, xi, va: (te[t], 0, h)),
        ],
        out_specs=pl.BlockSpec((TM, TH), lambda h, t, te, xi, va: (xi[t], h)),
    )
    return pl.pallas_call(
        _mlp1_kernel,
        grid_spec=grid_spec,
        out_shape=jax.ShapeDtypeStruct((NROWS, H), jnp.bfloat16),
        compiler_params=pltpu.CompilerParams(
            dimension_semantics=("arbitrary", "arbitrary")),
        interpret=interpret,
    )(texp, xsidx, valid, xs, w1, b1.reshape(E, 1, H))


DC = 512
ND = D // DC


def _mlp2_kernel(texp_ref, xsidx_ref, valid_ref,
                 hs_ref, w2_ref, b2_ref, ys_ref):
    t = pl.program_id(1)

    @pl.when(valid_ref[t] == 1)
    def _():
        hb = hs_ref[...]
        w2 = w2_ref[0].astype(jnp.bfloat16)
        ys_ref[...] = jax.lax.dot(
            hb, w2, preferred_element_type=jnp.float32) + b2_ref[0]


def _run_mlp2(hs, w2, b2, texp, xsidx, valid, *, interpret=False):
    grid_spec = pltpu.PrefetchScalarGridSpec(
        num_scalar_prefetch=3,
        grid=(ND, T),
        in_specs=[
            pl.BlockSpec((TM, H), lambda d, t, te, xi, va: (xi[t], 0)),
            pl.BlockSpec((1, H, DC), lambda d, t, te, xi, va: (te[t], 0, d)),
            pl.BlockSpec((1, 1, DC), lambda d, t, te, xi, va: (te[t], 0, d)),
        ],
        out_specs=pl.BlockSpec((TM, DC), lambda d, t, te, xi, va: (xi[t], d)),
    )
    return pl.pallas_call(
        _mlp2_kernel,
        grid_spec=grid_spec,
        out_shape=jax.ShapeDtypeStruct((NROWS, D), jnp.float32),
        compiler_params=pltpu.CompilerParams(
            dimension_semantics=("arbitrary", "arbitrary")),
        interpret=interpret,
    )(texp, xsidx, valid, hs, w2, b2.reshape(E, 1, D))


def _gather_sc(ys, q0, q1):
    mesh = plsc.VectorSubcoreMesh(core_axis_name="c", subcore_axis_name="s")
    out_t = (jax.ShapeDtypeStruct((B, D), ys.dtype),
             jax.ShapeDtypeStruct((B, D), ys.dtype))

    @functools.partial(
        pl.kernel, mesh=mesh, out_type=out_t,
        scratch_types=[
            pltpu.VMEM((_CH,), jnp.int32),
            pltpu.VMEM((_CH,), jnp.int32),
            pltpu.VMEM((_CH, D), ys.dtype),
            pltpu.VMEM((_CH, D), ys.dtype),
        ],
    )
    def gather_kernel(ys_hbm, q0_hbm, q1_hbm, o0_hbm, o1_hbm,
                      i0_v, i1_v, r0_v, r1_v):
        wid = jax.lax.axis_index("s") * _NC + jax.lax.axis_index("c")

        @pl.loop(0, _NCH)
        def _(c):
            j = wid * _NCH + c
            base = j * _CH
            pltpu.sync_copy(q0_hbm.at[j], i0_v)
            pltpu.sync_copy(q1_hbm.at[j], i1_v)
            pltpu.sync_copy(ys_hbm.at[i0_v], r0_v)
            pltpu.sync_copy(ys_hbm.at[i1_v], r1_v)
            pltpu.sync_copy(r0_v, o0_hbm.at[pl.ds(base, _CH)])
            pltpu.sync_copy(r1_v, o1_hbm.at[pl.ds(base, _CH)])

    return gather_kernel(ys, q0, q1)


_CB = 256


def _combine_kernel(y0_ref, y1_ref, g_ref, o_ref):
    g0 = g_ref[:, 0:1]
    g1 = g_ref[:, 1:2]
    o_ref[...] = y0_ref[...] * g0 + y1_ref[...] * g1


def _run_combine(y0, y1, gates, *, interpret=False):
    return pl.pallas_call(
        _combine_kernel,
        grid=(B // _CB,),
        in_specs=[
            pl.BlockSpec((_CB, D), lambda i: (i, 0)),
            pl.BlockSpec((_CB, D), lambda i: (i, 0)),
            pl.BlockSpec((_CB, K), lambda i: (i, 0)),
        ],
        out_specs=pl.BlockSpec((_CB, D), lambda i: (i, 0)),
        out_shape=jax.ShapeDtypeStruct((B, D), jnp.float32),
        interpret=interpret,
    )(y0, y1, gates)


def kernel(x, Wg, bg, W1, b1, W2, b2):
    gates, pos0, pos1, texp, xsidx, valid = _run_router(
        x, Wg.T, bg.reshape(1, E))
    p0 = pos0.reshape(_NW * _NCH, _CH)
    p1 = pos1.reshape(_NW * _NCH, _CH)
    x32 = jax.lax.bitcast_convert_type(
        x.astype(jnp.bfloat16).reshape(B, D // 2, 2), jnp.int32)
    xs32 = _dispatch_sc(x32, p0, p1)
    xs = jax.lax.bitcast_convert_type(xs32, jnp.bfloat16).reshape(NROWS, D)
    te, xi, va = texp.reshape(T), xsidx.reshape(T), valid.reshape(T)
    hs = _run_mlp1(xs, W1, b1, te, xi, va)
    ys = _run_mlp2(hs, W2, b2, te, xi, va)
    y0, y1 = _gather_sc(ys, p0, p1)
    return _run_combine(y0, y1, gates)


# speedup vs baseline: 1.5806x; 1.5806x over previous
import functools

import jax
import jax.numpy as jnp
from jax.experimental import pallas as pl
from jax.experimental.pallas import tpu as pltpu
from jax.experimental.pallas import tpu_sc as plsc

B = 2048
D = 2048
E = 8
K = 2
H = 4096

TM = 512
T = B * K // TM + E
NROWS = T * TM
TH = 1024
NH = H // TH

_NEG_INF = float("-inf")


def _router_kernel(x_ref, wgt_ref, bg_ref, gates_ref, pos0_ref, pos1_ref,
                   texp_ref, xsidx_ref, valid_ref):
    x = x_ref[...]
    logits = jax.lax.dot(x, wgt_ref[...],
                         preferred_element_type=jnp.float32)
    logits = logits + bg_ref[...]

    idx = jax.lax.broadcasted_iota(jnp.int32, (B, E), 1)
    m1 = jnp.max(logits, axis=1, keepdims=True)
    i1 = jnp.min(jnp.where(logits == m1, idx, E), axis=1, keepdims=True)
    masked = jnp.where(idx == i1, _NEG_INF, logits)
    m2 = jnp.max(masked, axis=1, keepdims=True)
    i2 = jnp.min(jnp.where(masked == m2, idx, E), axis=1, keepdims=True)

    e2 = jnp.exp(m2 - m1)
    denom = 1.0 + e2
    g1 = 1.0 / denom
    g2 = e2 / denom
    gates_ref[...] = jnp.concatenate([g1, g2], axis=1)

    oh1 = (idx == i1).astype(jnp.float32)
    oh2 = (idx == i2).astype(jnp.float32)
    mh = oh1 + oh2

    r_iota = jax.lax.broadcasted_iota(jnp.int32, (B, B), 0)
    c_iota = jax.lax.broadcasted_iota(jnp.int32, (B, B), 1)
    ltri = (c_iota < r_iota).astype(jnp.float32)
    csum = jax.lax.dot(ltri, mh, preferred_element_type=jnp.float32)

    counts = jnp.sum(mh, axis=0, keepdims=True)
    padded = jnp.ceil(counts / TM) * TM

    er = jax.lax.broadcasted_iota(jnp.int32, (E, E), 0)
    ec = jax.lax.broadcasted_iota(jnp.int32, (E, E), 1)
    utri = (er < ec).astype(jnp.float32)
    starts = jax.lax.dot(padded, utri,
                         preferred_element_type=jnp.float32)
    ends = starts + padded
    total = jnp.sum(padded, axis=1, keepdims=True)

    rank1 = jnp.sum(csum * oh1, axis=1, keepdims=True)
    rank2 = jnp.sum(csum * oh2, axis=1, keepdims=True)
    start1 = jnp.sum(starts * oh1, axis=1, keepdims=True)
    start2 = jnp.sum(starts * oh2, axis=1, keepdims=True)
    pos0_ref[...] = jnp.round(start1 + rank1).astype(jnp.int32)
    pos1_ref[...] = jnp.round(start2 + rank2).astype(jnp.int32)

    t_col = (jax.lax.broadcasted_iota(jnp.int32, (T, 1), 0) * TM
             ).astype(jnp.float32)
    t_cmp = jnp.sum((jnp.broadcast_to(ends, (T, E)) <=
                     jnp.broadcast_to(t_col, (T, E))).astype(jnp.int32),
                    axis=1, keepdims=True)
    texp_last = jnp.sum((ends <= (total - TM)).astype(jnp.int32),
                        axis=1, keepdims=True)
    texp_ref[...] = jnp.minimum(t_cmp, texp_last)
    n_last = jnp.round(total / TM).astype(jnp.int32) - 1
    t_idx = jax.lax.broadcasted_iota(jnp.int32, (T, 1), 0)
    xsidx_ref[...] = jnp.minimum(t_idx, n_last)
    valid_ref[...] = (t_idx <= n_last).astype(jnp.int32)


def _run_router(x, wgt, bg2d, *, interpret=False):
    out_shapes = (
        jax.ShapeDtypeStruct((B, K), jnp.float32),
        jax.ShapeDtypeStruct((B, 1), jnp.int32),
        jax.ShapeDtypeStruct((B, 1), jnp.int32),
        jax.ShapeDtypeStruct((T, 1), jnp.int32),
        jax.ShapeDtypeStruct((T, 1), jnp.int32),
        jax.ShapeDtypeStruct((T, 1), jnp.int32),
    )
    return pl.pallas_call(
        _router_kernel,
        out_shape=out_shapes,
        interpret=interpret,
    )(x, wgt, bg2d)


_NC = 2
_NS = 16
_NW = _NC * _NS
_CH = 16
_NCH = B // (_NW * _CH)


def _dispatch_sc(x, pos0, pos1):
    mesh = plsc.VectorSubcoreMesh(core_axis_name="c", subcore_axis_name="s")
    width = x.shape[1]

    @functools.partial(
        pl.kernel, mesh=mesh,
        out_type=jax.ShapeDtypeStruct((NROWS, width), x.dtype),
        scratch_types=[
            pltpu.VMEM((_CH,), jnp.int32),
            pltpu.VMEM((_CH,), jnp.int32),
            pltpu.VMEM((_CH, width), x.dtype),
        ],
    )
    def scatter_kernel(x_hbm, p0_hbm, p1_hbm, o_hbm, i0_v, i1_v, rows_v):
        wid = jax.lax.axis_index("s") * _NC + jax.lax.axis_index("c")

        @pl.loop(0, _NCH)
        def _(c):
            j = wid * _NCH + c
            base = j * _CH
            pltpu.sync_copy(p0_hbm.at[j], i0_v)
            pltpu.sync_copy(p1_hbm.at[j], i1_v)
            pltpu.sync_copy(x_hbm.at[pl.ds(base, _CH)], rows_v)
            pltpu.sync_copy(rows_v, o_hbm.at[i0_v])
            pltpu.sync_copy(rows_v, o_hbm.at[i1_v])

    return scatter_kernel(x, pos0, pos1)


def _mlp1_kernel(texp_ref, xsidx_ref, valid_ref,
                 xs_ref, w1_ref, b1_ref, hs_ref):
    t = pl.program_id(1)

    @pl.when(valid_ref[t] == 1)
    def _():
        xb = xs_ref[...].astype(jnp.bfloat16)
        w1 = w1_ref[0].astype(jnp.bfloat16)
        hb = jax.lax.dot(xb, w1, preferred_element_type=jnp.float32)
        hb = jnp.maximum(hb + b1_ref[0], 0.0)
        hs_ref[...] = hb.astype(jnp.bfloat16)


def _run_mlp1(xs, w1, b1, texp, xsidx, valid, *, interpret=False):
    grid_spec = pltpu.PrefetchScalarGridSpec(
        num_scalar_prefetch=3,
        grid=(NH, T),
        in_specs=[
            pl.BlockSpec((TM, D), lambda h, t, te, xi, va: (xi[t], 0)),
            pl.BlockSpec((1, D, TH), lambda h, t, te, xi, va: (te[t], 0, h)),
            pl.BlockSpec((1, 1, TH), lambda h, t, te, xi, va: (te[t], 0, h)),
        ],
        out_specs=pl.BlockSpec((TM, TH), lambda h, t, te, xi, va: (xi[t], h)),
    )
    return pl.pallas_call(
        _mlp1_kernel,
        grid_spec=grid_spec,
        out_shape=jax.ShapeDtypeStruct((NROWS, H), jnp.bfloat16),
        compiler_params=pltpu.CompilerParams(
            dimension_semantics=("arbitrary", "arbitrary")),
        interpret=interpret,
    )(texp, xsidx, valid, xs, w1, b1.reshape(E, 1, H))


DC = 512
ND = D // DC


def _mlp2_kernel(texp_ref, xsidx_ref, valid_ref,
                 hs_ref, w2_ref, b2_ref, ys_ref):
    t = pl.program_id(1)

    @pl.when(valid_ref[t] == 1)
    def _():
        hb = hs_ref[...]
        w2 = w2_ref[0].astype(jnp.bfloat16)
        ys_ref[...] = jax.lax.dot(
            hb, w2, preferred_element_type=jnp.float32) + b2_ref[0]


def _run_mlp2(hs, w2, b2, texp, xsidx, valid, *, interpret=False):
    grid_spec = pltpu.PrefetchScalarGridSpec(
        num_scalar_prefetch=3,
        grid=(ND, T),
        in_specs=[
            pl.BlockSpec((TM, H), lambda d, t, te, xi, va: (xi[t], 0)),
            pl.BlockSpec((1, H, DC), lambda d, t, te, xi, va: (te[t], 0, d)),
            pl.BlockSpec((1, 1, DC), lambda d, t, te, xi, va: (te[t], 0, d)),
        ],
        out_specs=pl.BlockSpec((TM, DC), lambda d, t, te, xi, va: (xi[t], d)),
    )
    return pl.pallas_call(
        _mlp2_kernel,
        grid_spec=grid_spec,
        out_shape=jax.ShapeDtypeStruct((NROWS, D), jnp.float32),
        compiler_params=pltpu.CompilerParams(
            dimension_semantics=("arbitrary", "arbitrary")),
        interpret=interpret,
    )(texp, xsidx, valid, hs, w2, b2.reshape(E, 1, D))


def _gather_sc(ys, q0, q1):
    mesh = plsc.VectorSubcoreMesh(core_axis_name="c", subcore_axis_name="s")
    out_t = (jax.ShapeDtypeStruct((B, D), ys.dtype),
             jax.ShapeDtypeStruct((B, D), ys.dtype))

    @functools.partial(
        pl.kernel, mesh=mesh, out_type=out_t,
        scratch_types=[
            pltpu.VMEM((_CH,), jnp.int32),
            pltpu.VMEM((_CH,), jnp.int32),
            pltpu.VMEM((_CH, D), ys.dtype),
            pltpu.VMEM((_CH, D), ys.dtype),
        ],
    )
    def gather_kernel(ys_hbm, q0_hbm, q1_hbm, o0_hbm, o1_hbm,
                      i0_v, i1_v, r0_v, r1_v):
        wid = jax.lax.axis_index("s") * _NC + jax.lax.axis_index("c")

        @pl.loop(0, _NCH)
        def _(c):
            j = wid * _NCH + c
            base = j * _CH
            pltpu.sync_copy(q0_hbm.at[j], i0_v)
            pltpu.sync_copy(q1_hbm.at[j], i1_v)
            pltpu.sync_copy(ys_hbm.at[i0_v], r0_v)
            pltpu.sync_copy(ys_hbm.at[i1_v], r1_v)
            pltpu.sync_copy(r0_v, o0_hbm.at[pl.ds(base, _CH)])
            pltpu.sync_copy(r1_v, o1_hbm.at[pl.ds(base, _CH)])

    return gather_kernel(ys, q0, q1)


_CB = 256


def _combine_kernel(y0_ref, y1_ref, g_ref, o_ref):
    g0 = g_ref[:, 0:1]
    g1 = g_ref[:, 1:2]
    o_ref[...] = y0_ref[...] * g0 + y1_ref[...] * g1


def _run_combine(y0, y1, gates, *, interpret=False):
    return pl.pallas_call(
        _combine_kernel,
        grid=(B // _CB,),
        in_specs=[
            pl.BlockSpec((_CB, D), lambda i: (i, 0)),
            pl.BlockSpec((_CB, D), lambda i: (i, 0)),
            pl.BlockSpec((_CB, K), lambda i: (i, 0)),
        ],
        out_specs=pl.BlockSpec((_CB, D), lambda i: (i, 0)),
        out_shape=jax.ShapeDtypeStruct((B, D), jnp.float32),
        interpret=interpret,
    )(y0, y1, gates)


def kernel(x, Wg, bg, W1, b1, W2, b2):
    gates, pos0, pos1, texp, xsidx, valid = _run_router(
        x, Wg.T, bg.reshape(1, E))
    p0 = pos0.reshape(_NW * _NCH, _CH)
    p1 = pos1.reshape(_NW * _NCH, _CH)
    xs = _dispatch_sc(x, p0, p1)
    te, xi, va = texp.reshape(T), xsidx.reshape(T), valid.reshape(T)
    hs = _run_mlp1(xs, W1, b1, te, xi, va)
    ys = _run_mlp2(hs, W2, b2, te, xi, va)
    y0, y1 = _gather_sc(ys, p0, p1)
    return _run_combine(y0, y1, gates)


# speedup vs baseline: 2.6006x; 1.6454x over previous
import functools

import jax
import jax.numpy as jnp
from jax.experimental import pallas as pl
from jax.experimental.pallas import tpu as pltpu
from jax.experimental.pallas import tpu_sc as plsc

B = 2048
D = 2048
E = 8
K = 2
H = 4096

TM = 576
T = -(-B * K // TM) + E
NROWS = T * TM
TH = 1024
NH = H // TH

_NEG_INF = float("-inf")


def _router_kernel(x_ref, wgt_ref, bg_ref, gates_ref, pos0_ref, pos1_ref,
                   texp_ref, xsidx_ref, valid_ref):
    x = x_ref[...]
    logits = jax.lax.dot(x, wgt_ref[...],
                         preferred_element_type=jnp.float32)
    logits = logits + bg_ref[...]

    idx = jax.lax.broadcasted_iota(jnp.int32, (B, E), 1)
    m1 = jnp.max(logits, axis=1, keepdims=True)
    i1 = jnp.min(jnp.where(logits == m1, idx, E), axis=1, keepdims=True)
    masked = jnp.where(idx == i1, _NEG_INF, logits)
    m2 = jnp.max(masked, axis=1, keepdims=True)
    i2 = jnp.min(jnp.where(masked == m2, idx, E), axis=1, keepdims=True)

    e2 = jnp.exp(m2 - m1)
    denom = 1.0 + e2
    g1 = 1.0 / denom
    g2 = e2 / denom
    gates_ref[...] = jnp.concatenate([g1, g2], axis=1)

    oh1 = (idx == i1).astype(jnp.float32)
    oh2 = (idx == i2).astype(jnp.float32)
    mh = oh1 + oh2

    r_iota = jax.lax.broadcasted_iota(jnp.int32, (B, B), 0)
    c_iota = jax.lax.broadcasted_iota(jnp.int32, (B, B), 1)
    ltri = (c_iota < r_iota).astype(jnp.float32)
    csum = jax.lax.dot(ltri, mh, preferred_element_type=jnp.float32)

    counts = jnp.sum(mh, axis=0, keepdims=True)
    padded = jnp.ceil(counts / TM) * TM

    er = jax.lax.broadcasted_iota(jnp.int32, (E, E), 0)
    ec = jax.lax.broadcasted_iota(jnp.int32, (E, E), 1)
    utri = (er < ec).astype(jnp.float32)
    starts = jax.lax.dot(padded, utri,
                         preferred_element_type=jnp.float32)
    ends = starts + padded
    total = jnp.sum(padded, axis=1, keepdims=True)

    rank1 = jnp.sum(csum * oh1, axis=1, keepdims=True)
    rank2 = jnp.sum(csum * oh2, axis=1, keepdims=True)
    start1 = jnp.sum(starts * oh1, axis=1, keepdims=True)
    start2 = jnp.sum(starts * oh2, axis=1, keepdims=True)
    pos0_ref[...] = jnp.round(start1 + rank1).astype(jnp.int32)
    pos1_ref[...] = jnp.round(start2 + rank2).astype(jnp.int32)

    t_col = (jax.lax.broadcasted_iota(jnp.int32, (T, 1), 0) * TM
             ).astype(jnp.float32)
    t_cmp = jnp.sum((jnp.broadcast_to(ends, (T, E)) <=
                     jnp.broadcast_to(t_col, (T, E))).astype(jnp.int32),
                    axis=1, keepdims=True)
    texp_last = jnp.sum((ends <= (total - TM)).astype(jnp.int32),
                        axis=1, keepdims=True)
    texp_ref[...] = jnp.minimum(t_cmp, texp_last)
    n_last = jnp.round(total / TM).astype(jnp.int32) - 1
    t_idx = jax.lax.broadcasted_iota(jnp.int32, (T, 1), 0)
    xsidx_ref[...] = jnp.minimum(t_idx, n_last)
    valid_ref[...] = (t_idx <= n_last).astype(jnp.int32)


def _run_router(x, wgt, bg2d, *, interpret=False):
    out_shapes = (
        jax.ShapeDtypeStruct((B, K), jnp.float32),
        jax.ShapeDtypeStruct((B, 1), jnp.int32),
        jax.ShapeDtypeStruct((B, 1), jnp.int32),
        jax.ShapeDtypeStruct((T, 1), jnp.int32),
        jax.ShapeDtypeStruct((T, 1), jnp.int32),
        jax.ShapeDtypeStruct((T, 1), jnp.int32),
    )
    return pl.pallas_call(
        _router_kernel,
        out_shape=out_shapes,
        interpret=interpret,
    )(x, wgt, bg2d)


_NC = 2
_NS = 16
_NW = _NC * _NS
_CH = 16
_NCH = B // (_NW * _CH)


def _dispatch_sc(x, pos0, pos1):
    mesh = plsc.VectorSubcoreMesh(core_axis_name="c", subcore_axis_name="s")
    width = x.shape[1]

    @functools.partial(
        pl.kernel, mesh=mesh,
        out_type=jax.ShapeDtypeStruct((NROWS, width), x.dtype),
        scratch_types=[
            pltpu.VMEM((_CH,), jnp.int32),
            pltpu.VMEM((_CH,), jnp.int32),
            pltpu.VMEM((_CH, width), x.dtype),
        ],
    )
    def scatter_kernel(x_hbm, p0_hbm, p1_hbm, o_hbm, i0_v, i1_v, rows_v):
        wid = jax.lax.axis_index("s") * _NC + jax.lax.axis_index("c")

        @pl.loop(0, _NCH)
        def _(c):
            j = wid * _NCH + c
            base = j * _CH
            pltpu.sync_copy(p0_hbm.at[j], i0_v)
            pltpu.sync_copy(p1_hbm.at[j], i1_v)
            pltpu.sync_copy(x_hbm.at[pl.ds(base, _CH)], rows_v)
            pltpu.sync_copy(rows_v, o_hbm.at[i0_v])
            pltpu.sync_copy(rows_v, o_hbm.at[i1_v])

    return scatter_kernel(x, pos0, pos1)


def _mlp_kernel(texp_ref, xsidx_ref, valid_ref,
                xs_ref, w1_ref, b1_ref, w2_ref, b2_ref, ys_ref):
    h = pl.program_id(1)
    t = pl.program_id(0)

    @pl.when(valid_ref[t] == 1)
    def _():
        xb = xs_ref[...]
        hb = jax.lax.dot(xb, w1_ref[0],
                         preferred_element_type=jnp.float32)
        hb = jnp.maximum(hb + b1_ref[0], 0.0)
        contrib = jax.lax.dot(hb, w2_ref[0],
                              preferred_element_type=jnp.float32)

        @pl.when(h == 0)
        def _():
            ys_ref[...] = contrib + b2_ref[0]

        @pl.when(h != 0)
        def _():
            ys_ref[...] += contrib


def _run_mlp(xs, w1, b1, w2, b2, texp, xsidx, valid, *, interpret=False):
    def sel_h(h, valid_ref, t):
        return jnp.where(valid_ref[t] == 1, h, NH - 1)

    grid_spec = pltpu.PrefetchScalarGridSpec(
        num_scalar_prefetch=3,
        grid=(T, NH),
        in_specs=[
            pl.BlockSpec((TM, D), lambda t, h, te, xi, va: (xi[t], 0)),
            pl.BlockSpec((1, D, TH),
                         lambda t, h, te, xi, va: (te[t], 0, sel_h(h, va, t))),
            pl.BlockSpec((1, 1, TH),
                         lambda t, h, te, xi, va: (te[t], 0, sel_h(h, va, t))),
            pl.BlockSpec((1, TH, D),
                         lambda t, h, te, xi, va: (te[t], sel_h(h, va, t), 0)),
            pl.BlockSpec((1, 1, D), lambda t, h, te, xi, va: (te[t], 0, 0)),
        ],
        out_specs=pl.BlockSpec((TM, D), lambda t, h, te, xi, va: (xi[t], 0)),
    )
    return pl.pallas_call(
        _mlp_kernel,
        grid_spec=grid_spec,
        out_shape=jax.ShapeDtypeStruct((NROWS, D), jnp.float32),
        compiler_params=pltpu.CompilerParams(
            dimension_semantics=("arbitrary", "arbitrary")),
        interpret=interpret,
    )(texp, xsidx, valid, xs, w1, b1.reshape(E, 1, H), w2,
      b2.reshape(E, 1, D))


def _gather_sc(ys, q0, q1):
    mesh = plsc.VectorSubcoreMesh(core_axis_name="c", subcore_axis_name="s")
    out_t = (jax.ShapeDtypeStruct((B, D), ys.dtype),
             jax.ShapeDtypeStruct((B, D), ys.dtype))

    @functools.partial(
        pl.kernel, mesh=mesh, out_type=out_t,
        scratch_types=[
            pltpu.VMEM((_CH,), jnp.int32),
            pltpu.VMEM((_CH,), jnp.int32),
            pltpu.VMEM((_CH, D), ys.dtype),
            pltpu.VMEM((_CH, D), ys.dtype),
        ],
    )
    def gather_kernel(ys_hbm, q0_hbm, q1_hbm, o0_hbm, o1_hbm,
                      i0_v, i1_v, r0_v, r1_v):
        wid = jax.lax.axis_index("s") * _NC + jax.lax.axis_index("c")

        @pl.loop(0, _NCH)
        def _(c):
            j = wid * _NCH + c
            base = j * _CH
            pltpu.sync_copy(q0_hbm.at[j], i0_v)
            pltpu.sync_copy(q1_hbm.at[j], i1_v)
            pltpu.sync_copy(ys_hbm.at[i0_v], r0_v)
            pltpu.sync_copy(ys_hbm.at[i1_v], r1_v)
            pltpu.sync_copy(r0_v, o0_hbm.at[pl.ds(base, _CH)])
            pltpu.sync_copy(r1_v, o1_hbm.at[pl.ds(base, _CH)])

    return gather_kernel(ys, q0, q1)


_CB = 256


def _combine_kernel(y0_ref, y1_ref, g_ref, o_ref):
    g0 = g_ref[:, 0:1]
    g1 = g_ref[:, 1:2]
    o_ref[...] = y0_ref[...] * g0 + y1_ref[...] * g1


def _run_combine(y0, y1, gates, *, interpret=False):
    return pl.pallas_call(
        _combine_kernel,
        grid=(B // _CB,),
        in_specs=[
            pl.BlockSpec((_CB, D), lambda i: (i, 0)),
            pl.BlockSpec((_CB, D), lambda i: (i, 0)),
            pl.BlockSpec((_CB, K), lambda i: (i, 0)),
        ],
        out_specs=pl.BlockSpec((_CB, D), lambda i: (i, 0)),
        out_shape=jax.ShapeDtypeStruct((B, D), jnp.float32),
        interpret=interpret,
    )(y0, y1, gates)


def kernel(x, Wg, bg, W1, b1, W2, b2):
    gates, pos0, pos1, texp, xsidx, valid = _run_router(
        x, Wg.T, bg.reshape(1, E))
    p0 = pos0.reshape(_NW * _NCH, _CH)
    p1 = pos1.reshape(_NW * _NCH, _CH)
    xs = _dispatch_sc(x, p0, p1)
    ys = _run_mlp(xs, W1, b1, W2, b2,
                  texp.reshape(T), xsidx.reshape(T), valid.reshape(T))
    y0, y1 = _gather_sc(ys, p0, p1)
    return _run_combine(y0, y1, gates)


# speedup vs baseline: 2.6045x; 1.0015x over previous
import functools

import jax
import jax.numpy as jnp
from jax.experimental import pallas as pl
from jax.experimental.pallas import tpu as pltpu
from jax.experimental.pallas import tpu_sc as plsc

B = 2048
D = 2048
E = 8
K = 2
H = 4096

TM = 576
T = -(-B * K // TM) + E
NROWS = T * TM
TH = 1024
NH = H // TH

_NEG_INF = float("-inf")


def _router_kernel(x_ref, wgt_ref, bg_ref, gates_ref, pos0_ref, pos1_ref,
                   texp_ref, xsidx_ref, valid_ref):
    x = x_ref[...]
    logits = jax.lax.dot(x, wgt_ref[...],
                         preferred_element_type=jnp.float32)
    logits = logits + bg_ref[...]

    idx = jax.lax.broadcasted_iota(jnp.int32, (B, E), 1)
    m1 = jnp.max(logits, axis=1, keepdims=True)
    i1 = jnp.min(jnp.where(logits == m1, idx, E), axis=1, keepdims=True)
    masked = jnp.where(idx == i1, _NEG_INF, logits)
    m2 = jnp.max(masked, axis=1, keepdims=True)
    i2 = jnp.min(jnp.where(masked == m2, idx, E), axis=1, keepdims=True)

    e2 = jnp.exp(m2 - m1)
    denom = 1.0 + e2
    g1 = 1.0 / denom
    g2 = e2 / denom
    gates_ref[...] = jnp.concatenate([g1, g2], axis=1)

    oh1 = (idx == i1).astype(jnp.float32)
    oh2 = (idx == i2).astype(jnp.float32)
    mh = oh1 + oh2

    r_iota = jax.lax.broadcasted_iota(jnp.int32, (B, B), 0)
    c_iota = jax.lax.broadcasted_iota(jnp.int32, (B, B), 1)
    ltri = (c_iota < r_iota).astype(jnp.float32)
    csum = jax.lax.dot(ltri, mh, preferred_element_type=jnp.float32)

    counts = jnp.sum(mh, axis=0, keepdims=True)
    padded = jnp.ceil(counts / TM) * TM

    er = jax.lax.broadcasted_iota(jnp.int32, (E, E), 0)
    ec = jax.lax.broadcasted_iota(jnp.int32, (E, E), 1)
    utri = (er < ec).astype(jnp.float32)
    starts = jax.lax.dot(padded, utri,
                         preferred_element_type=jnp.float32)
    ends = starts + padded
    total = jnp.sum(padded, axis=1, keepdims=True)

    rank1 = jnp.sum(csum * oh1, axis=1, keepdims=True)
    rank2 = jnp.sum(csum * oh2, axis=1, keepdims=True)
    start1 = jnp.sum(starts * oh1, axis=1, keepdims=True)
    start2 = jnp.sum(starts * oh2, axis=1, keepdims=True)
    pos0_ref[...] = jnp.round(start1 + rank1).astype(jnp.int32)
    pos1_ref[...] = jnp.round(start2 + rank2).astype(jnp.int32)

    t_col = (jax.lax.broadcasted_iota(jnp.int32, (T, 1), 0) * TM
             ).astype(jnp.float32)
    t_cmp = jnp.sum((jnp.broadcast_to(ends, (T, E)) <=
                     jnp.broadcast_to(t_col, (T, E))).astype(jnp.int32),
                    axis=1, keepdims=True)
    texp_last = jnp.sum((ends <= (total - TM)).astype(jnp.int32),
                        axis=1, keepdims=True)
    texp_ref[...] = jnp.minimum(t_cmp, texp_last)
    n_last = jnp.round(total / TM).astype(jnp.int32) - 1
    t_idx = jax.lax.broadcasted_iota(jnp.int32, (T, 1), 0)
    xsidx_ref[...] = jnp.minimum(t_idx, n_last)
    valid_ref[...] = (t_idx <= n_last).astype(jnp.int32)


def _run_router(x, wgt, bg2d, *, interpret=False):
    out_shapes = (
        jax.ShapeDtypeStruct((B, K), jnp.float32),
        jax.ShapeDtypeStruct((B, 1), jnp.int32),
        jax.ShapeDtypeStruct((B, 1), jnp.int32),
        jax.ShapeDtypeStruct((T, 1), jnp.int32),
        jax.ShapeDtypeStruct((T, 1), jnp.int32),
        jax.ShapeDtypeStruct((T, 1), jnp.int32),
    )
    return pl.pallas_call(
        _router_kernel,
        out_shape=out_shapes,
        interpret=interpret,
    )(x, wgt, bg2d)


_NC = 2
_NS = 16
_NW = _NC * _NS
_CH = 16
_NCH = B // (_NW * _CH)


def _dispatch_sc(x, pos0, pos1):
    mesh = plsc.VectorSubcoreMesh(core_axis_name="c", subcore_axis_name="s")
    width = x.shape[1]

    @functools.partial(
        pl.kernel, mesh=mesh,
        out_type=jax.ShapeDtypeStruct((NROWS, width), x.dtype),
        scratch_types=[
            pltpu.VMEM((_CH,), jnp.int32),
            pltpu.VMEM((_CH,), jnp.int32),
            pltpu.VMEM((_CH, width), x.dtype),
        ],
    )
    def scatter_kernel(x_hbm, p0_hbm, p1_hbm, o_hbm, i0_v, i1_v, rows_v):
        wid = jax.lax.axis_index("s") * _NC + jax.lax.axis_index("c")

        @pl.loop(0, _NCH)
        def _(c):
            j = wid * _NCH + c
            base = j * _CH
            pltpu.sync_copy(p0_hbm.at[j], i0_v)
            pltpu.sync_copy(p1_hbm.at[j], i1_v)
            pltpu.sync_copy(x_hbm.at[pl.ds(base, _CH)], rows_v)
            pltpu.sync_copy(rows_v, o_hbm.at[i0_v])
            pltpu.sync_copy(rows_v, o_hbm.at[i1_v])

    return scatter_kernel(x, pos0, pos1)


def _mlp_kernel(texp_ref, xsidx_ref, valid_ref,
                xs_ref, w1_ref, b1_ref, w2_ref, b2_ref, ys_ref):
    h = pl.program_id(1)
    t = pl.program_id(0)

    @pl.when(valid_ref[t] == 1)
    def _():
        xb = xs_ref[...].astype(jnp.bfloat16)
        hb = jax.lax.dot(xb, w1_ref[0].astype(jnp.bfloat16),
                         preferred_element_type=jnp.float32)
        hb = jnp.maximum(hb + b1_ref[0], 0.0)
        contrib = jax.lax.dot(hb.astype(jnp.bfloat16),
                              w2_ref[0].astype(jnp.bfloat16),
                              preferred_element_type=jnp.float32)

        @pl.when(h == 0)
        def _():
            ys_ref[...] = contrib + b2_ref[0]

        @pl.when(h != 0)
        def _():
            ys_ref[...] += contrib


def _run_mlp(xs, w1, b1, w2, b2, texp, xsidx, valid, *, interpret=False):
    def sel_h(h, valid_ref, t):
        return jnp.where(valid_ref[t] == 1, h, NH - 1)

    grid_spec = pltpu.PrefetchScalarGridSpec(
        num_scalar_prefetch=3,
        grid=(T, NH),
        in_specs=[
            pl.BlockSpec((TM, D), lambda t, h, te, xi, va: (xi[t], 0)),
            pl.BlockSpec((1, D, TH),
                         lambda t, h, te, xi, va: (te[t], 0, sel_h(h, va, t))),
            pl.BlockSpec((1, 1, TH),
                         lambda t, h, te, xi, va: (te[t], 0, sel_h(h, va, t))),
            pl.BlockSpec((1, TH, D),
                         lambda t, h, te, xi, va: (te[t], sel_h(h, va, t), 0)),
            pl.BlockSpec((1, 1, D), lambda t, h, te, xi, va: (te[t], 0, 0)),
        ],
        out_specs=pl.BlockSpec((TM, D), lambda t, h, te, xi, va: (xi[t], 0)),
    )
    return pl.pallas_call(
        _mlp_kernel,
        grid_spec=grid_spec,
        out_shape=jax.ShapeDtypeStruct((NROWS, D), jnp.float32),
        compiler_params=pltpu.CompilerParams(
            dimension_semantics=("arbitrary", "arbitrary")),
        interpret=interpret,
    )(texp, xsidx, valid, xs, w1, b1.reshape(E, 1, H), w2,
      b2.reshape(E, 1, D))


def _gather_sc(ys, q0, q1):
    mesh = plsc.VectorSubcoreMesh(core_axis_name="c", subcore_axis_name="s")
    out_t = (jax.ShapeDtypeStruct((B, D), ys.dtype),
             jax.ShapeDtypeStruct((B, D), ys.dtype))

    @functools.partial(
        pl.kernel, mesh=mesh, out_type=out_t,
        scratch_types=[
            pltpu.VMEM((_CH,), jnp.int32),
            pltpu.VMEM((_CH,), jnp.int32),
            pltpu.VMEM((_CH, D), ys.dtype),
            pltpu.VMEM((_CH, D), ys.dtype),
        ],
    )
    def gather_kernel(ys_hbm, q0_hbm, q1_hbm, o0_hbm, o1_hbm,
                      i0_v, i1_v, r0_v, r1_v):
        wid = jax.lax.axis_index("s") * _NC + jax.lax.axis_index("c")

        @pl.loop(0, _NCH)
        def _(c):
            j = wid * _NCH + c
            base = j * _CH
            pltpu.sync_copy(q0_hbm.at[j], i0_v)
            pltpu.sync_copy(q1_hbm.at[j], i1_v)
            pltpu.sync_copy(ys_hbm.at[i0_v], r0_v)
            pltpu.sync_copy(ys_hbm.at[i1_v], r1_v)
            pltpu.sync_copy(r0_v, o0_hbm.at[pl.ds(base, _CH)])
            pltpu.sync_copy(r1_v, o1_hbm.at[pl.ds(base, _CH)])

    return gather_kernel(ys, q0, q1)


_CB = 256


def _combine_kernel(y0_ref, y1_ref, g_ref, o_ref):
    g0 = g_ref[:, 0:1]
    g1 = g_ref[:, 1:2]
    o_ref[...] = y0_ref[...] * g0 + y1_ref[...] * g1


def _run_combine(y0, y1, gates, *, interpret=False):
    return pl.pallas_call(
        _combine_kernel,
        grid=(B // _CB,),
        in_specs=[
            pl.BlockSpec((_CB, D), lambda i: (i, 0)),
            pl.BlockSpec((_CB, D), lambda i: (i, 0)),
            pl.BlockSpec((_CB, K), lambda i: (i, 0)),
        ],
        out_specs=pl.BlockSpec((_CB, D), lambda i: (i, 0)),
        out_shape=jax.ShapeDtypeStruct((B, D), jnp.float32),
        interpret=interpret,
    )(y0, y1, gates)


def kernel(x, Wg, bg, W1, b1, W2, b2):
    gates, pos0, pos1, texp, xsidx, valid = _run_router(
        x, Wg.T, bg.reshape(1, E))
    p0 = pos0.reshape(_NW * _NCH, _CH)
    p1 = pos1.reshape(_NW * _NCH, _CH)
    xs = _dispatch_sc(x, p0, p1)
    ys = _run_mlp(xs, W1, b1, W2, b2,
                  texp.reshape(T), xsidx.reshape(T), valid.reshape(T))
    y0, y1 = _gather_sc(ys, p0, p1)
    return _run_combine(y0, y1, gates)
